# Initial kernel scaffold; baseline (speedup 1.0000x reference)
import jax
import jax.numpy as jnp
from jax.experimental import pallas as pl


def kernel(numbers, value_embs):
    raise NotImplementedError("write your pallas kernel here")


# speedup vs baseline: 3.7158x; 3.7158x over previous
import jax
import jax.numpy as jnp
import numpy as np
from jax.experimental import pallas as pl

HIDDEN = 128
NUM_EMB = 16
_POWERS = np.tile(
    np.array([[10.0 ** i for i in range(NUM_EMB)]], dtype=np.float32), (1, 8)
)
_RECIPS = np.float32(1.0) / _POWERS
_SCALES = _RECIPS * np.float32(0.0625)

_ROWS_PER_BLOCK = 8


def _tc_kernel(nums_ref, wtab_ref, pow_ref, rcp_ref, scl_ref, out_ref):
    rb = _ROWS_PER_BLOCK
    x = nums_ref[...].astype(jnp.float32)
    xb = jax.lax.broadcast_in_dim(x, (rb, 128, 128), (0, 1))
    xcol = xb.reshape(rb * 128, 128)
    pw = pow_ref[...]
    rc = rcp_ref[...]
    sc = scl_ref[...]
    q = jnp.floor(xcol * rc)
    r = xcol - q * pw
    r = jnp.where(r == pw, jnp.float32(0.0), r)
    coeff = jnp.abs(r) * sc
    out_ref[...] = jax.lax.dot_general(
        coeff, wtab_ref[...],
        dimension_numbers=(((1,), (0,)), ((), ())),
        preferred_element_type=jnp.float32,
    )


@jax.jit
def kernel(numbers, value_embs):
    b, l = numbers.shape
    n = b * l
    nums2d = numbers.reshape(n // 128, 128)
    wtab = jnp.tile(value_embs, (8, 1)) * (1.0 / 8.0)
    rb = _ROWS_PER_BLOCK
    grid = (n // 128) // rb
    out = pl.pallas_call(
        _tc_kernel,
        grid=(grid,),
        in_specs=[
            pl.BlockSpec((rb, 128), lambda i: (i, 0)),
            pl.BlockSpec((128, HIDDEN), lambda i: (0, 0)),
            pl.BlockSpec((1, 128), lambda i: (0, 0)),
            pl.BlockSpec((1, 128), lambda i: (0, 0)),
            pl.BlockSpec((1, 128), lambda i: (0, 0)),
        ],
        out_specs=pl.BlockSpec((rb * 128, HIDDEN), lambda i: (i, 0)),
        out_shape=jax.ShapeDtypeStruct((n, HIDDEN), jnp.float32),
    )(nums2d, wtab, jnp.asarray(_POWERS), jnp.asarray(_RECIPS),
      jnp.asarray(_SCALES))
    return out.reshape(b, l, HIDDEN)


# speedup vs baseline: 5.9021x; 1.5884x over previous
import jax
import jax.numpy as jnp
import numpy as np
from jax.experimental import pallas as pl

HIDDEN = 128
NUM_EMB = 16
_POWERS = np.tile(
    np.array([[10.0 ** i for i in range(NUM_EMB)]], dtype=np.float32), (1, 8)
)
_RECIPS = np.float32(1.0) / _POWERS
_SCALES = _RECIPS * np.float32(0.0625)

_ROWS_PER_BLOCK = 16


def _tc_kernel(nums_ref, wtab_ref, pow_ref, rcp_ref, scl_ref, out_ref):
    rb = _ROWS_PER_BLOCK
    x = nums_ref[...].astype(jnp.float32)
    xb = jax.lax.broadcast_in_dim(x, (rb, 128, 128), (0, 1))
    xcol = xb.reshape(rb * 128, 128)
    pw = pow_ref[...]
    rc = rcp_ref[...]
    sc = scl_ref[...]
    q = jnp.floor(xcol * rc)
    r = xcol - q * pw
    r = jnp.where(r == pw, jnp.float32(0.0), r)
    coeff = jnp.abs(r) * sc
    out_ref[...] = jax.lax.dot_general(
        coeff, wtab_ref[...],
        dimension_numbers=(((1,), (0,)), ((), ())),
        preferred_element_type=jnp.float32,
    )


@jax.jit
def kernel(numbers, value_embs):
    b, l = numbers.shape
    n = b * l
    nums2d = numbers.reshape(n // 128, 128)
    wtab = jnp.tile(value_embs, (8, 1)) * (1.0 / 8.0)
    rb = _ROWS_PER_BLOCK
    grid = (n // 128) // rb
    out = pl.pallas_call(
        _tc_kernel,
        grid=(grid,),
        in_specs=[
            pl.BlockSpec((rb, 128), lambda i: (i, 0)),
            pl.BlockSpec((128, HIDDEN), lambda i: (0, 0)),
            pl.BlockSpec((1, 128), lambda i: (0, 0)),
            pl.BlockSpec((1, 128), lambda i: (0, 0)),
            pl.BlockSpec((1, 128), lambda i: (0, 0)),
        ],
        out_specs=pl.BlockSpec((rb * 128, HIDDEN), lambda i: (i, 0)),
        out_shape=jax.ShapeDtypeStruct((n, HIDDEN), jnp.float32),
    )(nums2d, wtab, jnp.asarray(_POWERS), jnp.asarray(_RECIPS),
      jnp.asarray(_SCALES))
    return out.reshape(b, l, HIDDEN)


# speedup vs baseline: 8.6214x; 1.4608x over previous
import jax
import jax.numpy as jnp
import numpy as np
from jax.experimental import pallas as pl

HIDDEN = 128
NUM_EMB = 16
_POWERS = np.tile(
    np.array([[10.0 ** i for i in range(NUM_EMB)]], dtype=np.float32), (1, 8)
)
_RECIPS = np.float32(1.0) / _POWERS
_SCALES = _RECIPS * np.float32(0.0625)

_ROWS_PER_BLOCK = 32


def _tc_kernel(nums_ref, wtab_ref, pow_ref, rcp_ref, scl_ref, out_ref):
    rb = _ROWS_PER_BLOCK
    x = nums_ref[...].astype(jnp.float32)
    xb = jax.lax.broadcast_in_dim(x, (rb, 128, 128), (0, 1))
    xcol = xb.reshape(rb * 128, 128)
    pw = pow_ref[...]
    rc = rcp_ref[...]
    sc = scl_ref[...]
    q = jnp.floor(xcol * rc)
    r = xcol - q * pw
    r = jnp.where(r == pw, jnp.float32(0.0), r)
    coeff = jnp.abs(r) * sc
    out_ref[...] = jax.lax.dot_general(
        coeff, wtab_ref[...],
        dimension_numbers=(((1,), (0,)), ((), ())),
        preferred_element_type=jnp.float32,
    )


@jax.jit
def kernel(numbers, value_embs):
    b, l = numbers.shape
    n = b * l
    nums2d = numbers.reshape(n // 128, 128)
    wtab = jnp.tile(value_embs, (8, 1)) * (1.0 / 8.0)
    rb = _ROWS_PER_BLOCK
    grid = (n // 128) // rb
    out = pl.pallas_call(
        _tc_kernel,
        grid=(grid,),
        in_specs=[
            pl.BlockSpec((rb, 128), lambda i: (i, 0)),
            pl.BlockSpec((128, HIDDEN), lambda i: (0, 0)),
            pl.BlockSpec((1, 128), lambda i: (0, 0)),
            pl.BlockSpec((1, 128), lambda i: (0, 0)),
            pl.BlockSpec((1, 128), lambda i: (0, 0)),
        ],
        out_specs=pl.BlockSpec((rb * 128, HIDDEN), lambda i: (i, 0)),
        out_shape=jax.ShapeDtypeStruct((n, HIDDEN), jnp.float32),
    )(nums2d, wtab, jnp.asarray(_POWERS), jnp.asarray(_RECIPS),
      jnp.asarray(_SCALES))
    return out.reshape(b, l, HIDDEN)


# speedup vs baseline: 11.0234x; 1.2786x over previous
import jax
import jax.numpy as jnp
import numpy as np
from jax.experimental import pallas as pl

HIDDEN = 128
NUM_EMB = 16
_POWERS = np.tile(
    np.array([[10.0 ** i for i in range(NUM_EMB)]], dtype=np.float32), (1, 8)
)
_RECIPS = np.float32(1.0) / _POWERS
_SCALES = _RECIPS * np.float32(0.0625)

_ROWS_PER_BLOCK = 64


def _tc_kernel(nums_ref, wtab_ref, pow_ref, rcp_ref, scl_ref, out_ref):
    rb = _ROWS_PER_BLOCK
    x = nums_ref[...].astype(jnp.float32)
    xb = jax.lax.broadcast_in_dim(x, (rb, 128, 128), (0, 1))
    xcol = xb.reshape(rb * 128, 128)
    pw = pow_ref[...]
    rc = rcp_ref[...]
    sc = scl_ref[...]
    q = jnp.floor(xcol * rc)
    r = xcol - q * pw
    r = jnp.where(r == pw, jnp.float32(0.0), r)
    coeff = jnp.abs(r) * sc
    out_ref[...] = jax.lax.dot_general(
        coeff, wtab_ref[...],
        dimension_numbers=(((1,), (0,)), ((), ())),
        preferred_element_type=jnp.float32,
    )


@jax.jit
def kernel(numbers, value_embs):
    b, l = numbers.shape
    n = b * l
    nums2d = numbers.reshape(n // 128, 128)
    wtab = jnp.tile(value_embs, (8, 1)) * (1.0 / 8.0)
    rb = _ROWS_PER_BLOCK
    grid = (n // 128) // rb
    out = pl.pallas_call(
        _tc_kernel,
        grid=(grid,),
        in_specs=[
            pl.BlockSpec((rb, 128), lambda i: (i, 0)),
            pl.BlockSpec((128, HIDDEN), lambda i: (0, 0)),
            pl.BlockSpec((1, 128), lambda i: (0, 0)),
            pl.BlockSpec((1, 128), lambda i: (0, 0)),
            pl.BlockSpec((1, 128), lambda i: (0, 0)),
        ],
        out_specs=pl.BlockSpec((rb * 128, HIDDEN), lambda i: (i, 0)),
        out_shape=jax.ShapeDtypeStruct((n, HIDDEN), jnp.float32),
    )(nums2d, wtab, jnp.asarray(_POWERS), jnp.asarray(_RECIPS),
      jnp.asarray(_SCALES))
    return out.reshape(b, l, HIDDEN)


# speedup vs baseline: 12.5788x; 1.1411x over previous
import jax
import jax.numpy as jnp
import numpy as np
from jax.experimental import pallas as pl

HIDDEN = 128
NUM_EMB = 16
_POWERS = np.tile(
    np.array([[10.0 ** i for i in range(NUM_EMB)]], dtype=np.float32), (1, 8)
)
_RECIPS = np.float32(1.0) / _POWERS
_SCALES = _RECIPS * np.float32(0.0625)

_ROWS_PER_BLOCK = 128


def _tc_kernel(nums_ref, wtab_ref, pow_ref, rcp_ref, scl_ref, out_ref):
    rb = _ROWS_PER_BLOCK
    x = nums_ref[...].astype(jnp.float32)
    xb = jax.lax.broadcast_in_dim(x, (rb, 128, 128), (0, 1))
    xcol = xb.reshape(rb * 128, 128)
    pw = pow_ref[...]
    rc = rcp_ref[...]
    sc = scl_ref[...]
    q = jnp.floor(xcol * rc)
    r = xcol - q * pw
    r = jnp.where(r == pw, jnp.float32(0.0), r)
    coeff = jnp.abs(r) * sc
    out_ref[...] = jax.lax.dot_general(
        coeff, wtab_ref[...],
        dimension_numbers=(((1,), (0,)), ((), ())),
        preferred_element_type=jnp.float32,
    )


@jax.jit
def kernel(numbers, value_embs):
    b, l = numbers.shape
    n = b * l
    nums2d = numbers.reshape(n // 128, 128)
    wtab = jnp.tile(value_embs, (8, 1)) * (1.0 / 8.0)
    rb = _ROWS_PER_BLOCK
    grid = (n // 128) // rb
    out = pl.pallas_call(
        _tc_kernel,
        grid=(grid,),
        in_specs=[
            pl.BlockSpec((rb, 128), lambda i: (i, 0)),
            pl.BlockSpec((128, HIDDEN), lambda i: (0, 0)),
            pl.BlockSpec((1, 128), lambda i: (0, 0)),
            pl.BlockSpec((1, 128), lambda i: (0, 0)),
            pl.BlockSpec((1, 128), lambda i: (0, 0)),
        ],
        out_specs=pl.BlockSpec((rb * 128, HIDDEN), lambda i: (i, 0)),
        out_shape=jax.ShapeDtypeStruct((n, HIDDEN), jnp.float32),
    )(nums2d, wtab, jnp.asarray(_POWERS), jnp.asarray(_RECIPS),
      jnp.asarray(_SCALES))
    return out.reshape(b, l, HIDDEN)


# speedup vs baseline: 13.0149x; 1.0347x over previous
import jax
import jax.numpy as jnp
import numpy as np
from jax.experimental import pallas as pl

HIDDEN = 128
NUM_EMB = 16
_POWERS = np.tile(
    np.array([[10.0 ** i for i in range(NUM_EMB)]], dtype=np.float32), (1, 8)
)
_RECIPS = np.float32(1.0) / _POWERS
_SCALES = _RECIPS * np.float32(0.0625)

_ROWS_PER_BLOCK = 200


def _tc_kernel(nums_ref, wtab_ref, pow_ref, rcp_ref, scl_ref, out_ref):
    rb = _ROWS_PER_BLOCK
    x = nums_ref[...].astype(jnp.float32)
    xb = jax.lax.broadcast_in_dim(x, (rb, 128, 128), (0, 1))
    xcol = xb.reshape(rb * 128, 128)
    pw = pow_ref[...]
    rc = rcp_ref[...]
    sc = scl_ref[...]
    q = jnp.floor(xcol * rc)
    r = xcol - q * pw
    r = jnp.where(r == pw, jnp.float32(0.0), r)
    coeff = jnp.abs(r) * sc
    out_ref[...] = jax.lax.dot_general(
        coeff, wtab_ref[...],
        dimension_numbers=(((1,), (0,)), ((), ())),
        preferred_element_type=jnp.float32,
    )


@jax.jit
def kernel(numbers, value_embs):
    b, l = numbers.shape
    n = b * l
    nums2d = numbers.reshape(n // 128, 128)
    wtab = jnp.tile(value_embs, (8, 1)) * (1.0 / 8.0)
    rb = _ROWS_PER_BLOCK
    grid = (n // 128) // rb
    out = pl.pallas_call(
        _tc_kernel,
        grid=(grid,),
        in_specs=[
            pl.BlockSpec((rb, 128), lambda i: (i, 0)),
            pl.BlockSpec((128, HIDDEN), lambda i: (0, 0)),
            pl.BlockSpec((1, 128), lambda i: (0, 0)),
            pl.BlockSpec((1, 128), lambda i: (0, 0)),
            pl.BlockSpec((1, 128), lambda i: (0, 0)),
        ],
        out_specs=pl.BlockSpec((rb * 128, HIDDEN), lambda i: (i, 0)),
        out_shape=jax.ShapeDtypeStruct((n, HIDDEN), jnp.float32),
    )(nums2d, wtab, jnp.asarray(_POWERS), jnp.asarray(_RECIPS),
      jnp.asarray(_SCALES))
    return out.reshape(b, l, HIDDEN)


# speedup vs baseline: 13.0240x; 1.0007x over previous
import jax
import jax.numpy as jnp
import numpy as np
from jax.experimental import pallas as pl

HIDDEN = 128
NUM_EMB = 16
_POWERS = np.tile(
    np.array([[10.0 ** i for i in range(NUM_EMB)]], dtype=np.float32), (1, 8)
)
_RECIPS = np.float32(1.0) / _POWERS
_SCALES = _RECIPS * np.float32(0.0625)

_ROWS_PER_BLOCK = 200


def _tc_kernel(nums_ref, wtab_ref, pow_ref, rcp_ref, out_ref):
    rb = _ROWS_PER_BLOCK
    x = nums_ref[...].astype(jnp.float32)
    xb = jax.lax.broadcast_in_dim(x, (rb, 128, 128), (0, 1))
    xcol = xb.reshape(rb * 128, 128)
    pw = pow_ref[...]
    rc = rcp_ref[...]
    q = jnp.floor(xcol * rc)
    r = xcol - q * pw
    r = jnp.where(r == pw, jnp.float32(0.0), r)
    coeff = jnp.abs(r)
    out_ref[...] = jax.lax.dot_general(
        coeff, wtab_ref[...],
        dimension_numbers=(((1,), (0,)), ((), ())),
        preferred_element_type=jnp.float32,
    )


@jax.jit
def kernel(numbers, value_embs):
    b, l = numbers.shape
    n = b * l
    nums2d = numbers.reshape(n // 128, 128)
    wtab = jnp.tile(value_embs, (8, 1)) * (jnp.asarray(_SCALES).T * (1.0 / 8.0))
    rb = _ROWS_PER_BLOCK
    grid = (n // 128) // rb
    out = pl.pallas_call(
        _tc_kernel,
        grid=(grid,),
        in_specs=[
            pl.BlockSpec((rb, 128), lambda i: (i, 0)),
            pl.BlockSpec((128, HIDDEN), lambda i: (0, 0)),
            pl.BlockSpec((1, 128), lambda i: (0, 0)),
            pl.BlockSpec((1, 128), lambda i: (0, 0)),
        ],
        out_specs=pl.BlockSpec((rb * 128, HIDDEN), lambda i: (i, 0)),
        out_shape=jax.ShapeDtypeStruct((n, HIDDEN), jnp.float32),
    )(nums2d, wtab, jnp.asarray(_POWERS), jnp.asarray(_RECIPS))
    return out.reshape(b, l, HIDDEN)
